# baseline (speedup 1.0000x reference)
import functools

import jax
import jax.numpy as jnp
from jax import lax
from jax.experimental import pallas as pl
from jax.experimental.pallas import tpu as pltpu
from jax.experimental.pallas import tpu_sc as plsc

N_NODES = 100000
NH = 7
D = 128
LANES = 16
DREGS = D // LANES
CHUNK = 16
ROWS_G = CHUNK * (NH - 1)
NC = 2
NS = 16
NW = NC * NS
NCHUNKS = N_NODES // CHUNK
CH_PER_W = ((-(-NCHUNKS // NW)) + 7) // 8 * 8
NCH_PAD = CH_PER_W * NW
NBUF = 4


def _make_nh_sum():
    mesh = plsc.VectorSubcoreMesh(core_axis_name="c", subcore_axis_name="s")

    @functools.partial(
        pl.kernel,
        mesh=mesh,
        out_type=jax.ShapeDtypeStruct((NW, 8, LANES), jnp.float32),
        scratch_types=(
            [pltpu.VMEM((CH_PER_W, ROWS_G), jnp.int32)]
            + [pltpu.VMEM((ROWS_G, D), jnp.float32)] * NBUF
            + [pltpu.VMEM((CHUNK, D), jnp.float32)] * NBUF
            + [pltpu.VMEM((8, LANES), jnp.float32)]
            + [pltpu.SemaphoreType.DMA] * (2 * NBUF)
        ),
    )
    def nh_sum(table3, idx, out, idx_v, *rest):
        table = table3.at[0]
        nbufs = rest[0:NBUF]
        cbufs = rest[NBUF:2 * NBUF]
        accv = rest[2 * NBUF]
        nsems = rest[2 * NBUF + 1:3 * NBUF + 1]
        csems = rest[3 * NBUF + 1:4 * NBUF + 1]

        wid = lax.axis_index("s") * NC + lax.axis_index("c")
        base_chunk = wid * CH_PER_W
        pltpu.sync_copy(idx.at[pl.ds(base_chunk, CH_PER_W)], idx_v)

        def cbase(g):
            return jnp.minimum((base_chunk + g) * CHUNK, N_NODES - CHUNK)

        def gather_start(g, b):
            pltpu.async_copy(table.at[idx_v.at[g]], nbufs[b], nsems[b])
            pltpu.async_copy(table.at[pl.ds(cbase(g), CHUNK)],
                             cbufs[b], csems[b])

        def gather_wait(g, b):
            pltpu.make_async_copy(table.at[idx_v.at[g]],
                                  nbufs[b], nsems[b]).wait()
            pltpu.make_async_copy(table.at[pl.ds(cbase(g), CHUNK)],
                                  cbufs[b], csems[b]).wait()

        def chunk_sum(nbuf, cbuf, acc):
            def node_body(n, accs):
                b = n * (NH - 1)
                ctr = [cbuf[n, pl.ds(d * LANES, LANES)] for d in range(DREGS)]
                new = list(accs)
                for j in range(NH - 1):
                    for d in range(DREGS):
                        nb = nbuf[b + j, pl.ds(d * LANES, LANES)]
                        new[d] = new[d] + jnp.abs(ctr[d] - nb)
                return tuple(new)

            zeros = tuple(jnp.zeros((LANES,), jnp.float32) for _ in range(DREGS))
            accs = lax.fori_loop(0, CHUNK, node_body, zeros)
            s = accs[0]
            for d in range(1, DREGS):
                s = s + accs[d]
            return acc + s

        for b in range(NBUF - 1):
            gather_start(b, b)

        def outer(i, acc):
            g_base = NBUF * i
            for b in range(NBUF):
                g = g_base + b
                gather_wait(g, b)
                nxt = (b + NBUF - 1) % NBUF

                @pl.when(g + NBUF - 1 < CH_PER_W)
                def _():
                    gather_start(g + NBUF - 1, nxt)

                acc = chunk_sum(nbufs[b], cbufs[b], acc)
            return acc

        acc = lax.fori_loop(0, CH_PER_W // NBUF, outer,
                            jnp.zeros((LANES,), jnp.float32))
        zero = jnp.zeros((LANES,), jnp.float32)
        accv[0, :] = acc
        for r in range(1, 8):
            accv[r, :] = zero
        pltpu.sync_copy(accv, out.at[wid])

    return nh_sum


_nh_sum = _make_nh_sum()


def kernel(output, adjc):
    idx = adjc[:, 1:].reshape(NCHUNKS, ROWS_G)
    tail = jnp.arange(N_NODES - CHUNK, N_NODES, dtype=jnp.int32)
    pad_row = jnp.repeat(tail, NH - 1).reshape(1, ROWS_G)
    pad = jnp.broadcast_to(pad_row, (NCH_PAD - NCHUNKS, ROWS_G))
    idx = jnp.concatenate([idx, pad], axis=0)
    partials = _nh_sum(output, idx)
    total = jnp.sum(partials)
    count = output.shape[0] * N_NODES * (NH - 1) * D
    return jnp.sqrt(total / count)


# speedup vs baseline: 1.2538x; 1.2538x over previous
import functools

import jax
import jax.numpy as jnp
from jax import lax
from jax.experimental import pallas as pl
from jax.experimental.pallas import tpu as pltpu
from jax.experimental.pallas import tpu_sc as plsc

N_NODES = 100000
NH = 7
NB = NH - 1
D = 128
LANES = 16
DREGS = D // LANES
CHUNK = 16
ROWS_G = CHUNK * NB
NC = 2
NS = 16
NW = NC * NS
NCHUNKS = N_NODES // CHUNK
CH_PER_W = ((-(-NCHUNKS // NW)) + 7) // 8 * 8
NCH_PAD = CH_PER_W * NW
NODES_PW = CH_PER_W * CHUNK
N_PAD = NCH_PAD * CHUNK
NBUF = 4


def _make_nh_sum():
    mesh = plsc.VectorSubcoreMesh(core_axis_name="c", subcore_axis_name="s")

    @functools.partial(
        pl.kernel,
        mesh=mesh,
        out_type=jax.ShapeDtypeStruct((NW, 8, LANES), jnp.float32),
        scratch_types=(
            [pltpu.VMEM((NB, NODES_PW), jnp.int32)]
            + [pltpu.VMEM((ROWS_G,), jnp.int32)] * NBUF
            + [pltpu.VMEM((ROWS_G, D), jnp.float32)] * NBUF
            + [pltpu.VMEM((CHUNK, D), jnp.float32)] * NBUF
            + [pltpu.VMEM((8, LANES), jnp.float32)]
            + [pltpu.SemaphoreType.DMA] * (2 * NBUF)
        ),
    )
    def nh_sum(table3, idxt, out, idx_v, *rest):
        table = table3.at[0]
        stages = rest[0:NBUF]
        nbufs = rest[NBUF:2 * NBUF]
        cbufs = rest[2 * NBUF:3 * NBUF]
        accv = rest[3 * NBUF]
        nsems = rest[3 * NBUF + 1:4 * NBUF + 1]
        csems = rest[4 * NBUF + 1:5 * NBUF + 1]

        wid = lax.axis_index("s") * NC + lax.axis_index("c")
        base_chunk = wid * CH_PER_W
        for j in range(NB):
            pltpu.sync_copy(idxt.at[j, pl.ds(base_chunk * CHUNK, NODES_PW)],
                            idx_v.at[j])

        def cbase(g):
            return jnp.minimum((base_chunk + g) * CHUNK, N_NODES - CHUNK)

        def gather_start(g, b):
            for j in range(NB):
                stages[b][pl.ds(j * LANES, LANES)] = (
                    idx_v[j, pl.ds(g * CHUNK, LANES)])
            pltpu.async_copy(table.at[stages[b]], nbufs[b], nsems[b])
            pltpu.async_copy(table.at[pl.ds(cbase(g), CHUNK)],
                             cbufs[b], csems[b])

        def gather_wait(g, b):
            pltpu.make_async_copy(table.at[stages[b]],
                                  nbufs[b], nsems[b]).wait()
            pltpu.make_async_copy(table.at[pl.ds(cbase(g), CHUNK)],
                                  cbufs[b], csems[b]).wait()

        def chunk_sum(nbuf, cbuf, acc):
            def node_body(n, accs):
                ctr = [cbuf[n, pl.ds(d * LANES, LANES)] for d in range(DREGS)]
                new = list(accs)
                for j in range(NB):
                    for d in range(DREGS):
                        nb = nbuf[j * LANES + n, pl.ds(d * LANES, LANES)]
                        new[d] = new[d] + jnp.abs(ctr[d] - nb)
                return tuple(new)

            zeros = tuple(jnp.zeros((LANES,), jnp.float32) for _ in range(DREGS))
            accs = lax.fori_loop(0, CHUNK, node_body, zeros)
            s = accs[0]
            for d in range(1, DREGS):
                s = s + accs[d]
            return acc + s

        for b in range(NBUF - 1):
            gather_start(b, b)

        def outer(i, acc):
            g_base = NBUF * i
            for b in range(NBUF):
                g = g_base + b
                gather_wait(g, b)
                nxt = (b + NBUF - 1) % NBUF

                @pl.when(g + NBUF - 1 < CH_PER_W)
                def _():
                    gather_start(g + NBUF - 1, nxt)

                acc = chunk_sum(nbufs[b], cbufs[b], acc)
            return acc

        acc = lax.fori_loop(0, CH_PER_W // NBUF, outer,
                            jnp.zeros((LANES,), jnp.float32))
        zero = jnp.zeros((LANES,), jnp.float32)
        accv[0, :] = acc
        for r in range(1, 8):
            accv[r, :] = zero
        pltpu.sync_copy(accv, out.at[wid])

    return nh_sum


_nh_sum = _make_nh_sum()


def kernel(output, adjc):
    idxt = adjc.T[1:]
    pad_col = (N_NODES - CHUNK) + (jnp.arange(N_PAD - N_NODES,
                                              dtype=jnp.int32) % CHUNK)
    pad = jnp.broadcast_to(pad_col[None, :], (NB, N_PAD - N_NODES))
    idxt = jnp.concatenate([idxt, pad], axis=1)
    partials = _nh_sum(output, idxt)
    total = jnp.sum(partials)
    count = output.shape[0] * N_NODES * NB * D
    return jnp.sqrt(total / count)


# speedup vs baseline: 1.3591x; 1.0840x over previous
import functools

import jax
import jax.numpy as jnp
from jax import lax
from jax.experimental import pallas as pl
from jax.experimental.pallas import tpu as pltpu
from jax.experimental.pallas import tpu_sc as plsc

N_NODES = 100000
NH = 7
NB = NH - 1
D = 128
LANES = 16
DREGS = D // LANES
CHUNK = 16
ROWS_G = CHUNK * NB
NC = 2
NS = 16
NW = NC * NS
NCHUNKS = N_NODES // CHUNK
CH_C0 = 232
CH_C1 = 168
NCH_PAD = NS * (CH_C0 + CH_C1)
CH_MAX = max(CH_C0, CH_C1)
NODES_PW = CH_MAX * CHUNK
N_PAD = NCH_PAD * CHUNK
N_PAD_IDX = (NS - 1) * CH_C1 * CHUNK + NS * CH_C0 * CHUNK + NODES_PW
NBUF = 4


def _make_nh_sum():
    mesh = plsc.VectorSubcoreMesh(core_axis_name="c", subcore_axis_name="s")

    @functools.partial(
        pl.kernel,
        mesh=mesh,
        out_type=jax.ShapeDtypeStruct((NW, 8, LANES), jnp.float32),
        scratch_types=(
            [pltpu.VMEM((NB, NODES_PW), jnp.int32)]
            + [pltpu.VMEM((ROWS_G,), jnp.int32)] * NBUF
            + [pltpu.VMEM((ROWS_G, D), jnp.float32)] * NBUF
            + [pltpu.VMEM((CHUNK, D), jnp.float32)] * NBUF
            + [pltpu.VMEM((8, LANES), jnp.float32)]
            + [pltpu.SemaphoreType.DMA] * (2 * NBUF)
        ),
    )
    def nh_sum(table3, idxt, out, idx_v, *rest):
        table = table3.at[0]
        stages = rest[0:NBUF]
        nbufs = rest[NBUF:2 * NBUF]
        cbufs = rest[2 * NBUF:3 * NBUF]
        accv = rest[3 * NBUF]
        nsems = rest[3 * NBUF + 1:4 * NBUF + 1]
        csems = rest[4 * NBUF + 1:5 * NBUF + 1]

        core = lax.axis_index("c")
        sub = lax.axis_index("s")
        wid = sub * NC + core
        base_chunk = jnp.where(core == 0, sub * CH_C0,
                               NS * CH_C0 + sub * CH_C1)
        my_ch = jnp.where(core == 0, CH_C0, CH_C1)
        for j in range(NB):
            pltpu.sync_copy(idxt.at[j, pl.ds(base_chunk * CHUNK, NODES_PW)],
                            idx_v.at[j])

        def cbase(g):
            return jnp.minimum((base_chunk + g) * CHUNK, N_NODES - CHUNK)

        def gather_start(g, b):
            for j in range(NB):
                stages[b][pl.ds(j * LANES, LANES)] = (
                    idx_v[j, pl.ds(g * CHUNK, LANES)])
            pltpu.async_copy(table.at[stages[b]], nbufs[b], nsems[b])
            pltpu.async_copy(table.at[pl.ds(cbase(g), CHUNK)],
                             cbufs[b], csems[b])

        def gather_wait(g, b):
            pltpu.make_async_copy(table.at[stages[b]],
                                  nbufs[b], nsems[b]).wait()
            pltpu.make_async_copy(table.at[pl.ds(cbase(g), CHUNK)],
                                  cbufs[b], csems[b]).wait()

        def chunk_sum(nbuf, cbuf, acc):
            def node_body(n, accs):
                ctr = [cbuf[n, pl.ds(d * LANES, LANES)] for d in range(DREGS)]
                new = list(accs)
                for j in range(NB):
                    for d in range(DREGS):
                        nb = nbuf[j * LANES + n, pl.ds(d * LANES, LANES)]
                        new[d] = new[d] + jnp.abs(ctr[d] - nb)
                return tuple(new)

            zeros = tuple(jnp.zeros((LANES,), jnp.float32) for _ in range(DREGS))
            accs = lax.fori_loop(0, CHUNK, node_body, zeros)
            s = accs[0]
            for d in range(1, DREGS):
                s = s + accs[d]
            return acc + s

        for b in range(NBUF - 1):
            gather_start(b, b)

        def outer(i, acc):
            g_base = NBUF * i
            for b in range(NBUF):
                g = g_base + b
                gather_wait(g, b)
                nxt = (b + NBUF - 1) % NBUF

                @pl.when(g + NBUF - 1 < my_ch)
                def _():
                    gather_start(g + NBUF - 1, nxt)

                acc = chunk_sum(nbufs[b], cbufs[b], acc)
            return acc

        acc = lax.fori_loop(0, my_ch // NBUF, outer,
                            jnp.zeros((LANES,), jnp.float32))
        zero = jnp.zeros((LANES,), jnp.float32)
        accv[0, :] = acc
        for r in range(1, 8):
            accv[r, :] = zero
        pltpu.sync_copy(accv, out.at[wid])

    return nh_sum


_nh_sum = _make_nh_sum()


def kernel(output, adjc):
    idxt = adjc.T[1:]
    pad_col = (N_NODES - CHUNK) + (jnp.arange(N_PAD_IDX - N_NODES,
                                              dtype=jnp.int32) % CHUNK)
    pad = jnp.broadcast_to(pad_col[None, :], (NB, N_PAD_IDX - N_NODES))
    idxt = jnp.concatenate([idxt, pad], axis=1)
    partials = _nh_sum(output, idxt)
    total = jnp.sum(partials)
    count = output.shape[0] * N_NODES * NB * D
    return jnp.sqrt(total / count)


# speedup vs baseline: 1.4165x; 1.0422x over previous
import functools

import jax
import jax.numpy as jnp
from jax import lax
from jax.experimental import pallas as pl
from jax.experimental.pallas import tpu as pltpu
from jax.experimental.pallas import tpu_sc as plsc

N_NODES = 100000
NH = 7
NB = NH - 1
D = 128
LANES = 16
DREGS = D // LANES
CHUNK = 16
ROWS_G = CHUNK * NB
NC = 2
NS = 16
NW = NC * NS
NCHUNKS = N_NODES // CHUNK
CH_C0 = 240
CH_C1 = 160
NCH_PAD = NS * (CH_C0 + CH_C1)
CH_MAX = max(CH_C0, CH_C1)
NODES_PW = CH_MAX * CHUNK
N_PAD = NCH_PAD * CHUNK
_SLAB_END = (NS - 1) * CH_C1 * CHUNK + NS * CH_C0 * CHUNK + NODES_PW
N_PAD_IDX = -(-_SLAB_END // 1024) * 1024
NBUF = 4


def _make_nh_sum():
    mesh = plsc.VectorSubcoreMesh(core_axis_name="c", subcore_axis_name="s")

    @functools.partial(
        pl.kernel,
        mesh=mesh,
        out_type=jax.ShapeDtypeStruct((NW, 8, LANES), jnp.float32),
        scratch_types=(
            [pltpu.VMEM((NB, NODES_PW), jnp.int32)]
            + [pltpu.VMEM((ROWS_G,), jnp.int32)] * NBUF
            + [pltpu.VMEM((ROWS_G, D), jnp.float32)] * NBUF
            + [pltpu.VMEM((CHUNK, D), jnp.float32)] * NBUF
            + [pltpu.VMEM((8, LANES), jnp.float32)]
            + [pltpu.SemaphoreType.DMA] * (2 * NBUF)
        ),
    )
    def nh_sum(table3, idxt, out, idx_v, *rest):
        table = table3.at[0]
        stages = rest[0:NBUF]
        nbufs = rest[NBUF:2 * NBUF]
        cbufs = rest[2 * NBUF:3 * NBUF]
        accv = rest[3 * NBUF]
        nsems = rest[3 * NBUF + 1:4 * NBUF + 1]
        csems = rest[4 * NBUF + 1:5 * NBUF + 1]

        core = lax.axis_index("c")
        sub = lax.axis_index("s")
        wid = sub * NC + core
        base_chunk = jnp.where(core == 0, sub * CH_C0,
                               NS * CH_C0 + sub * CH_C1)
        my_ch = jnp.where(core == 0, CH_C0, CH_C1)
        for j in range(NB):
            pltpu.sync_copy(idxt.at[j, pl.ds(base_chunk * CHUNK, NODES_PW)],
                            idx_v.at[j])

        def cbase(g):
            return jnp.minimum((base_chunk + g) * CHUNK, N_NODES - CHUNK)

        def gather_start(g, b):
            for j in range(NB):
                stages[b][pl.ds(j * LANES, LANES)] = (
                    idx_v[j, pl.ds(g * CHUNK, LANES)])
            pltpu.async_copy(table.at[stages[b]], nbufs[b], nsems[b])
            pltpu.async_copy(table.at[pl.ds(cbase(g), CHUNK)],
                             cbufs[b], csems[b])

        def gather_wait(g, b):
            pltpu.make_async_copy(table.at[stages[b]],
                                  nbufs[b], nsems[b]).wait()
            pltpu.make_async_copy(table.at[pl.ds(cbase(g), CHUNK)],
                                  cbufs[b], csems[b]).wait()

        def chunk_sum(nbuf, cbuf, acc):
            def node_body(n, accs):
                ctr = [cbuf[n, pl.ds(d * LANES, LANES)] for d in range(DREGS)]
                new = list(accs)
                for j in range(NB):
                    for d in range(DREGS):
                        nb = nbuf[j * LANES + n, pl.ds(d * LANES, LANES)]
                        new[d] = new[d] + jnp.abs(ctr[d] - nb)
                return tuple(new)

            zeros = tuple(jnp.zeros((LANES,), jnp.float32) for _ in range(DREGS))
            accs = lax.fori_loop(0, CHUNK, node_body, zeros)
            s = accs[0]
            for d in range(1, DREGS):
                s = s + accs[d]
            return acc + s

        for b in range(NBUF - 1):
            gather_start(b, b)

        def outer(i, acc):
            g_base = NBUF * i
            for b in range(NBUF):
                g = g_base + b
                gather_wait(g, b)
                nxt = (b + NBUF - 1) % NBUF

                @pl.when(g + NBUF - 1 < my_ch)
                def _():
                    gather_start(g + NBUF - 1, nxt)

                acc = chunk_sum(nbufs[b], cbufs[b], acc)
            return acc

        acc = lax.fori_loop(0, my_ch // NBUF, outer,
                            jnp.zeros((LANES,), jnp.float32))
        zero = jnp.zeros((LANES,), jnp.float32)
        accv[0, :] = acc
        for r in range(1, 8):
            accv[r, :] = zero
        pltpu.sync_copy(accv, out.at[wid])

    return nh_sum


_nh_sum = _make_nh_sum()


def kernel(output, adjc):
    idxt = adjc.T[1:]
    pad_col = (N_NODES - CHUNK) + (jnp.arange(N_PAD_IDX - N_NODES,
                                              dtype=jnp.int32) % CHUNK)
    pad = jnp.broadcast_to(pad_col[None, :], (NB, N_PAD_IDX - N_NODES))
    idxt = jnp.concatenate([idxt, pad], axis=1)
    partials = _nh_sum(output, idxt)
    total = jnp.sum(partials)
    count = output.shape[0] * N_NODES * NB * D
    return jnp.sqrt(total / count)
